# Initial kernel scaffold; baseline (speedup 1.0000x reference)
import jax
import jax.numpy as jnp
from jax.experimental import pallas as pl


def kernel(x, edge_index, Wl1, bl1, Wr1, br1, att1, b1, Wl2, bl2, Wr2, br2, att2, b2, Wlin, blin):
    raise NotImplementedError("write your pallas kernel here")


# speedup vs baseline: 8.3529x; 8.3529x over previous
import functools

import jax
import jax.numpy as jnp
from jax import lax
from jax.experimental import pallas as pl
from jax.experimental.pallas import tpu as pltpu
from jax.experimental.pallas import tpu_sc as plsc

_N = 10000
_E = 320000
_H = 128
_NP = 10240
_CHUNK = 128
_NC = 2
_NS = 16
_NW = _NC * _NS
_CHUNKS_TOTAL = _E // _CHUNK
_ITERS = (_CHUNKS_TOTAL + _NW - 1) // _NW

_sc_mesh = plsc.VectorSubcoreMesh(
    core_axis_name="c", subcore_axis_name="s", num_cores=_NC)

_GATHER_DNUMS = lax.GatherDimensionNumbers(
    offset_dims=(), collapsed_slice_dims=(0,), start_index_map=(0,))


def _lane_gather(v, idx):
    return lax.gather(
        v, idx[:, None], _GATHER_DNUMS, slice_sizes=(1,),
        mode=lax.GatherScatterMode.PROMISE_IN_BOUNDS)


@functools.partial(
    pl.kernel,
    mesh=_sc_mesh,
    out_type=[
        jax.ShapeDtypeStruct((_NC, _N, _H), jnp.float32),
        jax.ShapeDtypeStruct((_NC, _NS, _NP), jnp.float32),
    ],
    scratch_types=[
        pltpu.VMEM((_CHUNK,), jnp.int32),
        pltpu.VMEM((_CHUNK,), jnp.int32),
        pltpu.VMEM((_CHUNK, _H), jnp.float32),
        pltpu.VMEM((_CHUNK, _H), jnp.float32),
        pltpu.VMEM((_H,), jnp.float32),
        pltpu.VMEM((_NP,), jnp.float32),
        pltpu.VMEM_SHARED((_N, _H), jnp.float32),
        pltpu.SemaphoreType.DMA,
        pltpu.SemaphoreType.DMA,
    ],
)
def _edge_pass(xl_hbm, xr_hbm, src_hbm, dst_hbm, att_hbm, zeros_hbm,
               feat_hbm, den_hbm, srcv, dstv, xlv, xrv, attv,
               denomv, accsh, sem1, sem2):
    cid = lax.axis_index("c")
    sid = lax.axis_index("s")

    @pl.when(sid == 0)
    def _():
        pltpu.sync_copy(zeros_hbm, accsh)

    pltpu.sync_copy(att_hbm, attv)

    def zero_body(i, carry):
        denomv[pl.ds(i * 16, 16)] = jnp.zeros((16,), jnp.float32)
        return carry

    lax.fori_loop(0, _NP // 16, zero_body, 0)
    plsc.subcore_barrier()

    wid = sid * _NC + cid

    def chunk_body(i, carry):
        chunk_id = i * _NW + wid

        @pl.when(chunk_id < _CHUNKS_TOTAL)
        def _():
            e0 = chunk_id * _CHUNK
            pltpu.sync_copy(src_hbm.at[pl.ds(e0, _CHUNK)], srcv)
            pltpu.sync_copy(dst_hbm.at[pl.ds(e0, _CHUNK)], dstv)
            g1 = pltpu.async_copy(xl_hbm.at[srcv], xlv, sem1)
            g2 = pltpu.async_copy(xr_hbm.at[dstv], xrv, sem2)
            g1.wait()
            g2.wait()

            def group_body(g, gcarry):
                dvec = dstv[pl.ds(g * 16, 16)]
                lanes = lax.iota(jnp.int32, 16)
                lane0 = lanes == 0
                for l in range(16):
                    b = g * 16 + l
                    acc = jnp.zeros((16,), jnp.float32)
                    for hc in range(_H // 16):
                        t = (xlv[b, pl.ds(hc * 16, 16)]
                             + xrv[b, pl.ds(hc * 16, 16)])
                        t = jnp.maximum(t, 0.2 * t)
                        acc = acc + t * attv[pl.ds(hc * 16, 16)]
                    for sh in (8, 4, 2, 1):
                        acc = acc + _lane_gather(acc, (lanes + sh) & 15)
                    a = jnp.exp(acc)
                    for hc in range(_H // 16):
                        xlv[b, pl.ds(hc * 16, 16)] = (
                            xlv[b, pl.ds(hc * 16, 16)] * a)
                    di = dvec[l]
                    dval = denomv[pl.ds(di, 16)]
                    denomv[pl.ds(di, 16)] = dval + jnp.where(lane0, a, 0.0)
                return gcarry

            lax.fori_loop(0, _CHUNK // 16, group_body, 0)
            pltpu.sync_copy(xlv, accsh.at[dstv], add=True)

        return carry

    lax.fori_loop(0, _ITERS, chunk_body, 0)

    pltpu.sync_copy(denomv, den_hbm.at[cid, sid])
    plsc.subcore_barrier()

    @pl.when(sid == 0)
    def _():
        pltpu.sync_copy(accsh, feat_hbm.at[cid])


def _proj_body(x_ref, wl_ref, bl_ref, wr_ref, br_ref, xl_ref, xr_ref):
    x = x_ref[...]
    cdims = (((1,), (1,)), ((), ()))
    xl_ref[...] = (
        lax.dot_general(x, wl_ref[...], cdims,
                        preferred_element_type=jnp.float32) + bl_ref[...])
    xr_ref[...] = (
        lax.dot_general(x, wr_ref[...], cdims,
                        preferred_element_type=jnp.float32) + br_ref[...])


def _project(x, wl, bl, wr, br):
    blk = 1000
    grid = _N // blk
    return pl.pallas_call(
        _proj_body,
        grid=(grid,),
        in_specs=[
            pl.BlockSpec((blk, _H), lambda i: (i, 0)),
            pl.BlockSpec((_H, _H), lambda i: (0, 0)),
            pl.BlockSpec((1, _H), lambda i: (0, 0)),
            pl.BlockSpec((_H, _H), lambda i: (0, 0)),
            pl.BlockSpec((1, _H), lambda i: (0, 0)),
        ],
        out_specs=[
            pl.BlockSpec((blk, _H), lambda i: (i, 0)),
            pl.BlockSpec((blk, _H), lambda i: (i, 0)),
        ],
        out_shape=[
            jax.ShapeDtypeStruct((_N, _H), jnp.float32),
            jax.ShapeDtypeStruct((_N, _H), jnp.float32),
        ],
    )(x, wl, bl.reshape(1, _H), wr, br.reshape(1, _H))


def _den_col(d, ones):
    return lax.dot_general(d, ones, (((1,), (0,)), ((), ())),
                           preferred_element_type=jnp.float32)


def _mid_body(a0_ref, a1_ref, d_ref, ones_ref, b1_ref, wl_ref, bl_ref,
              wr_ref, br_ref, h_ref, xl_ref, xr_ref):
    num = a0_ref[...] + a1_ref[...]
    den = _den_col(d_ref[...], ones_ref[...])
    h = jnp.maximum(num / (den + 1e-16) + b1_ref[...], 0.0)
    h_ref[...] = h
    cdims = (((1,), (1,)), ((), ()))
    xl_ref[...] = (
        lax.dot_general(h, wl_ref[...], cdims,
                        preferred_element_type=jnp.float32) + bl_ref[...])
    xr_ref[...] = (
        lax.dot_general(h, wr_ref[...], cdims,
                        preferred_element_type=jnp.float32) + br_ref[...])


def _mid_stage(feat, den, ones, b1, wl2, bl2, wr2, br2):
    blk = 1000
    grid = _N // blk
    return pl.pallas_call(
        _mid_body,
        grid=(grid,),
        in_specs=[
            pl.BlockSpec((blk, _H), lambda i: (i, 0)),
            pl.BlockSpec((blk, _H), lambda i: (i, 0)),
            pl.BlockSpec((blk, _NW), lambda i: (i, 0)),
            pl.BlockSpec((_NW, 1), lambda i: (0, 0)),
            pl.BlockSpec((1, _H), lambda i: (0, 0)),
            pl.BlockSpec((_H, _H), lambda i: (0, 0)),
            pl.BlockSpec((1, _H), lambda i: (0, 0)),
            pl.BlockSpec((_H, _H), lambda i: (0, 0)),
            pl.BlockSpec((1, _H), lambda i: (0, 0)),
        ],
        out_specs=[
            pl.BlockSpec((blk, _H), lambda i: (i, 0)),
            pl.BlockSpec((blk, _H), lambda i: (i, 0)),
            pl.BlockSpec((blk, _H), lambda i: (i, 0)),
        ],
        out_shape=[
            jax.ShapeDtypeStruct((_N, _H), jnp.float32),
            jax.ShapeDtypeStruct((_N, _H), jnp.float32),
            jax.ShapeDtypeStruct((_N, _H), jnp.float32),
        ],
    )(feat[0], feat[1], den, ones, b1.reshape(1, _H), wl2,
      bl2.reshape(1, _H), wr2, br2.reshape(1, _H))


def _final_body(h_ref, a0_ref, a1_ref, d_ref, ones_ref, b2_ref, wlin_ref,
                blin_ref, out_ref):
    num = a0_ref[...] + a1_ref[...]
    den = _den_col(d_ref[...], ones_ref[...])
    h2 = num / (den + 1e-16) + b2_ref[...]
    hf = h_ref[...] + h2
    cdims = (((1,), (1,)), ((), ()))
    logits = (
        lax.dot_general(hf, wlin_ref[...], cdims,
                        preferred_element_type=jnp.float32) + blin_ref[...])
    m = jnp.max(logits, axis=1, keepdims=True)
    z = logits - m
    lse = jnp.log(jnp.sum(jnp.exp(z), axis=1, keepdims=True))
    out_ref[...] = z - lse


def _final_stage(h, feat, den, ones, b2, wlin, blin):
    blk = 1000
    grid = _N // blk
    c = wlin.shape[0]
    return pl.pallas_call(
        _final_body,
        grid=(grid,),
        in_specs=[
            pl.BlockSpec((blk, _H), lambda i: (i, 0)),
            pl.BlockSpec((blk, _H), lambda i: (i, 0)),
            pl.BlockSpec((blk, _H), lambda i: (i, 0)),
            pl.BlockSpec((blk, _NW), lambda i: (i, 0)),
            pl.BlockSpec((_NW, 1), lambda i: (0, 0)),
            pl.BlockSpec((1, _H), lambda i: (0, 0)),
            pl.BlockSpec((c, _H), lambda i: (0, 0)),
            pl.BlockSpec((1, c), lambda i: (0, 0)),
        ],
        out_specs=pl.BlockSpec((blk, c), lambda i: (i, 0)),
        out_shape=jax.ShapeDtypeStruct((_N, c), jnp.float32),
    )(h, feat[0], feat[1], den, ones, b2.reshape(1, _H), wlin,
      blin.reshape(1, -1))


def kernel(x, edge_index, Wl1, bl1, Wr1, br1, att1, b1,
           Wl2, bl2, Wr2, br2, att2, b2, Wlin, blin):
    src = edge_index[0]
    dst = edge_index[1]
    zeros = jnp.zeros((_N, _H), jnp.float32)
    ones = jnp.ones((_NW, 1), jnp.float32)

    xl1, xr1 = _project(x, Wl1, bl1, Wr1, br1)
    feat1, den1 = _edge_pass(xl1, xr1, src, dst, att1, zeros)
    h, xl2, xr2 = _mid_stage(feat1, den1.reshape(_NW, _NP).T, ones,
                             b1, Wl2, bl2, Wr2, br2)
    feat2, den2 = _edge_pass(xl2, xr2, src, dst, att2, zeros)
    out = _final_stage(h, feat2, den2.reshape(_NW, _NP).T, ones,
                       b2, Wlin, blin)
    return (out, edge_index)


# speedup vs baseline: 9.0354x; 1.0817x over previous
import functools

import jax
import jax.numpy as jnp
from jax import lax
from jax.experimental import pallas as pl
from jax.experimental.pallas import tpu as pltpu
from jax.experimental.pallas import tpu_sc as plsc

_N = 10000
_E = 320000
_H = 128
_NP = 10240
_CHUNK = 128
_NC = 2
_NS = 16
_NW = _NC * _NS
_CHUNKS_TOTAL = _E // _CHUNK
_ITERS = (_CHUNKS_TOTAL + _NW - 1) // _NW

_sc_mesh = plsc.VectorSubcoreMesh(
    core_axis_name="c", subcore_axis_name="s", num_cores=_NC)

_GATHER_DNUMS = lax.GatherDimensionNumbers(
    offset_dims=(), collapsed_slice_dims=(0,), start_index_map=(0,))


def _lane_gather(v, idx):
    return lax.gather(
        v, idx[:, None], _GATHER_DNUMS, slice_sizes=(1,),
        mode=lax.GatherScatterMode.PROMISE_IN_BOUNDS)


@functools.partial(
    pl.kernel,
    mesh=_sc_mesh,
    out_type=[
        jax.ShapeDtypeStruct((_NC, _N, _H), jnp.float32),
        jax.ShapeDtypeStruct((_NC, _NS, _NP), jnp.float32),
    ],
    scratch_types=[
        pltpu.VMEM((_CHUNK,), jnp.int32),
        pltpu.VMEM((_CHUNK,), jnp.int32),
        pltpu.VMEM((_CHUNK, _H), jnp.float32),
        pltpu.VMEM((_CHUNK, _H), jnp.float32),
        pltpu.VMEM((_H,), jnp.float32),
        pltpu.VMEM((_NP,), jnp.float32),
        pltpu.VMEM_SHARED((_N, _H), jnp.float32),
        pltpu.SemaphoreType.DMA,
        pltpu.SemaphoreType.DMA,
    ],
)
def _edge_pass(xl_hbm, xr_hbm, src_hbm, dst_hbm, att_hbm, zeros_hbm,
               feat_hbm, den_hbm, srcv, dstv, xlv, xrv, attv,
               denomv, accsh, sem1, sem2):
    cid = lax.axis_index("c")
    sid = lax.axis_index("s")

    @pl.when(sid == 0)
    def _():
        pltpu.sync_copy(zeros_hbm, accsh)

    pltpu.sync_copy(att_hbm, attv)

    def zero_body(i, carry):
        denomv[pl.ds(i * 16, 16)] = jnp.zeros((16,), jnp.float32)
        return carry

    lax.fori_loop(0, _NP // 16, zero_body, 0)
    plsc.subcore_barrier()

    wid = sid * _NC + cid

    def chunk_body(i, carry):
        chunk_id = i * _NW + wid

        @pl.when(chunk_id < _CHUNKS_TOTAL)
        def _():
            e0 = chunk_id * _CHUNK
            pltpu.sync_copy(src_hbm.at[pl.ds(e0, _CHUNK)], srcv)
            pltpu.sync_copy(dst_hbm.at[pl.ds(e0, _CHUNK)], dstv)
            g1 = pltpu.async_copy(xl_hbm.at[srcv], xlv, sem1)
            g2 = pltpu.async_copy(xr_hbm.at[dstv], xrv, sem2)
            g1.wait()
            g2.wait()

            def group_body(g, gcarry):
                dvec = dstv[pl.ds(g * 16, 16)]
                lanes = lax.iota(jnp.int32, 16)
                lane0 = lanes == 0
                avec = jnp.zeros((16,), jnp.float32)
                for l in range(16):
                    b = g * 16 + l
                    acc = jnp.zeros((16,), jnp.float32)
                    xl_regs = []
                    for hc in range(_H // 16):
                        xl = xlv[b, pl.ds(hc * 16, 16)]
                        xl_regs.append(xl)
                        t = xl + xrv[b, pl.ds(hc * 16, 16)]
                        t = jnp.maximum(t, 0.2 * t)
                        acc = acc + t * attv[pl.ds(hc * 16, 16)]
                    for sh in (8, 4, 2, 1):
                        acc = acc + _lane_gather(acc, (lanes + sh) & 15)
                    a = jnp.exp(acc)
                    for hc in range(_H // 16):
                        xlv[b, pl.ds(hc * 16, 16)] = xl_regs[hc] * a
                    avec = jnp.where(lanes == l, a, avec)
                for l in range(16):
                    di = dvec[l]
                    al = _lane_gather(avec, jnp.full((16,), l, jnp.int32))
                    dval = denomv[pl.ds(di, 16)]
                    denomv[pl.ds(di, 16)] = (
                        dval + jnp.where(lane0, al, 0.0))
                return gcarry

            lax.fori_loop(0, _CHUNK // 16, group_body, 0)
            pltpu.sync_copy(xlv, accsh.at[dstv], add=True)

        return carry

    lax.fori_loop(0, _ITERS, chunk_body, 0)

    pltpu.sync_copy(denomv, den_hbm.at[cid, sid])
    plsc.subcore_barrier()

    @pl.when(sid == 0)
    def _():
        pltpu.sync_copy(accsh, feat_hbm.at[cid])


def _proj_body(x_ref, wl_ref, bl_ref, wr_ref, br_ref, xl_ref, xr_ref):
    x = x_ref[...]
    cdims = (((1,), (1,)), ((), ()))
    xl_ref[...] = (
        lax.dot_general(x, wl_ref[...], cdims,
                        preferred_element_type=jnp.float32) + bl_ref[...])
    xr_ref[...] = (
        lax.dot_general(x, wr_ref[...], cdims,
                        preferred_element_type=jnp.float32) + br_ref[...])


def _project(x, wl, bl, wr, br):
    blk = 1000
    grid = _N // blk
    return pl.pallas_call(
        _proj_body,
        grid=(grid,),
        in_specs=[
            pl.BlockSpec((blk, _H), lambda i: (i, 0)),
            pl.BlockSpec((_H, _H), lambda i: (0, 0)),
            pl.BlockSpec((1, _H), lambda i: (0, 0)),
            pl.BlockSpec((_H, _H), lambda i: (0, 0)),
            pl.BlockSpec((1, _H), lambda i: (0, 0)),
        ],
        out_specs=[
            pl.BlockSpec((blk, _H), lambda i: (i, 0)),
            pl.BlockSpec((blk, _H), lambda i: (i, 0)),
        ],
        out_shape=[
            jax.ShapeDtypeStruct((_N, _H), jnp.float32),
            jax.ShapeDtypeStruct((_N, _H), jnp.float32),
        ],
    )(x, wl, bl.reshape(1, _H), wr, br.reshape(1, _H))


def _den_col(d, ones):
    return lax.dot_general(d, ones, (((1,), (0,)), ((), ())),
                           preferred_element_type=jnp.float32)


def _mid_body(a0_ref, a1_ref, d_ref, ones_ref, b1_ref, wl_ref, bl_ref,
              wr_ref, br_ref, h_ref, xl_ref, xr_ref):
    num = a0_ref[...] + a1_ref[...]
    den = _den_col(d_ref[...], ones_ref[...])
    h = jnp.maximum(num / (den + 1e-16) + b1_ref[...], 0.0)
    h_ref[...] = h
    cdims = (((1,), (1,)), ((), ()))
    xl_ref[...] = (
        lax.dot_general(h, wl_ref[...], cdims,
                        preferred_element_type=jnp.float32) + bl_ref[...])
    xr_ref[...] = (
        lax.dot_general(h, wr_ref[...], cdims,
                        preferred_element_type=jnp.float32) + br_ref[...])


def _mid_stage(feat, den, ones, b1, wl2, bl2, wr2, br2):
    blk = 1000
    grid = _N // blk
    return pl.pallas_call(
        _mid_body,
        grid=(grid,),
        in_specs=[
            pl.BlockSpec((blk, _H), lambda i: (i, 0)),
            pl.BlockSpec((blk, _H), lambda i: (i, 0)),
            pl.BlockSpec((blk, _NW), lambda i: (i, 0)),
            pl.BlockSpec((_NW, 1), lambda i: (0, 0)),
            pl.BlockSpec((1, _H), lambda i: (0, 0)),
            pl.BlockSpec((_H, _H), lambda i: (0, 0)),
            pl.BlockSpec((1, _H), lambda i: (0, 0)),
            pl.BlockSpec((_H, _H), lambda i: (0, 0)),
            pl.BlockSpec((1, _H), lambda i: (0, 0)),
        ],
        out_specs=[
            pl.BlockSpec((blk, _H), lambda i: (i, 0)),
            pl.BlockSpec((blk, _H), lambda i: (i, 0)),
            pl.BlockSpec((blk, _H), lambda i: (i, 0)),
        ],
        out_shape=[
            jax.ShapeDtypeStruct((_N, _H), jnp.float32),
            jax.ShapeDtypeStruct((_N, _H), jnp.float32),
            jax.ShapeDtypeStruct((_N, _H), jnp.float32),
        ],
    )(feat[0], feat[1], den, ones, b1.reshape(1, _H), wl2,
      bl2.reshape(1, _H), wr2, br2.reshape(1, _H))


def _final_body(h_ref, a0_ref, a1_ref, d_ref, ones_ref, b2_ref, wlin_ref,
                blin_ref, out_ref):
    num = a0_ref[...] + a1_ref[...]
    den = _den_col(d_ref[...], ones_ref[...])
    h2 = num / (den + 1e-16) + b2_ref[...]
    hf = h_ref[...] + h2
    cdims = (((1,), (1,)), ((), ()))
    logits = (
        lax.dot_general(hf, wlin_ref[...], cdims,
                        preferred_element_type=jnp.float32) + blin_ref[...])
    m = jnp.max(logits, axis=1, keepdims=True)
    z = logits - m
    lse = jnp.log(jnp.sum(jnp.exp(z), axis=1, keepdims=True))
    out_ref[...] = z - lse


def _final_stage(h, feat, den, ones, b2, wlin, blin):
    blk = 1000
    grid = _N // blk
    c = wlin.shape[0]
    return pl.pallas_call(
        _final_body,
        grid=(grid,),
        in_specs=[
            pl.BlockSpec((blk, _H), lambda i: (i, 0)),
            pl.BlockSpec((blk, _H), lambda i: (i, 0)),
            pl.BlockSpec((blk, _H), lambda i: (i, 0)),
            pl.BlockSpec((blk, _NW), lambda i: (i, 0)),
            pl.BlockSpec((_NW, 1), lambda i: (0, 0)),
            pl.BlockSpec((1, _H), lambda i: (0, 0)),
            pl.BlockSpec((c, _H), lambda i: (0, 0)),
            pl.BlockSpec((1, c), lambda i: (0, 0)),
        ],
        out_specs=pl.BlockSpec((blk, c), lambda i: (i, 0)),
        out_shape=jax.ShapeDtypeStruct((_N, c), jnp.float32),
    )(h, feat[0], feat[1], den, ones, b2.reshape(1, _H), wlin,
      blin.reshape(1, -1))


def kernel(x, edge_index, Wl1, bl1, Wr1, br1, att1, b1,
           Wl2, bl2, Wr2, br2, att2, b2, Wlin, blin):
    src = edge_index[0]
    dst = edge_index[1]
    zeros = jnp.zeros((_N, _H), jnp.float32)
    ones = jnp.ones((_NW, 1), jnp.float32)

    xl1, xr1 = _project(x, Wl1, bl1, Wr1, br1)
    feat1, den1 = _edge_pass(xl1, xr1, src, dst, att1, zeros)
    h, xl2, xr2 = _mid_stage(feat1, den1.reshape(_NW, _NP).T, ones,
                             b1, Wl2, bl2, Wr2, br2)
    feat2, den2 = _edge_pass(xl2, xr2, src, dst, att2, zeros)
    out = _final_stage(h, feat2, den2.reshape(_NW, _NP).T, ones,
                       b2, Wlin, blin)
    return (out, edge_index)


# speedup vs baseline: 12.3423x; 1.3660x over previous
import functools

import jax
import jax.numpy as jnp
from jax import lax
from jax.experimental import pallas as pl
from jax.experimental.pallas import tpu as pltpu
from jax.experimental.pallas import tpu_sc as plsc

_N = 10000
_E = 320000
_H = 128
_NP = 10240
_CHUNK = 128
_NC = 2
_NS = 16
_NW = _NC * _NS
_CHUNKS_TOTAL = _E // _CHUNK
_ITERS = (_CHUNKS_TOTAL + _NW - 1) // _NW

_sc_mesh = plsc.VectorSubcoreMesh(
    core_axis_name="c", subcore_axis_name="s", num_cores=_NC)

_GATHER_DNUMS = lax.GatherDimensionNumbers(
    offset_dims=(), collapsed_slice_dims=(0,), start_index_map=(0,))


def _lane_gather(v, idx):
    return lax.gather(
        v, idx[:, None], _GATHER_DNUMS, slice_sizes=(1,),
        mode=lax.GatherScatterMode.PROMISE_IN_BOUNDS)


@functools.partial(
    pl.kernel,
    mesh=_sc_mesh,
    out_type=[
        jax.ShapeDtypeStruct((_NC, _N, _H), jnp.float32),
        jax.ShapeDtypeStruct((_NC, _NS, _NP), jnp.float32),
    ],
    scratch_types=[
        pltpu.VMEM((_CHUNK,), jnp.int32),
        pltpu.VMEM((_CHUNK,), jnp.int32),
        pltpu.VMEM((_CHUNK, _H), jnp.float32),
        pltpu.VMEM((_CHUNK, _H), jnp.float32),
        pltpu.VMEM((_H,), jnp.float32),
        pltpu.VMEM((_NP,), jnp.float32),
        pltpu.VMEM_SHARED((_N, _H), jnp.float32),
        pltpu.SemaphoreType.DMA,
        pltpu.SemaphoreType.DMA,
    ],
)
def _edge_pass(xl_hbm, xr_hbm, src_hbm, dst_hbm, att_hbm, zeros_hbm,
               feat_hbm, den_hbm, srcv, dstv, xlv, xrv, attv,
               denomv, accsh, sem1, sem2):
    cid = lax.axis_index("c")
    sid = lax.axis_index("s")

    @pl.when(sid == 0)
    def _():
        pltpu.sync_copy(zeros_hbm, accsh)

    pltpu.sync_copy(att_hbm, attv)

    def zero_body(i, carry):
        denomv[pl.ds(i * 16, 16)] = jnp.zeros((16,), jnp.float32)
        return carry

    lax.fori_loop(0, _NP // 16, zero_body, 0)
    plsc.subcore_barrier()

    wid = sid * _NC + cid
    att_regs = [attv[pl.ds(hc * 16, 16)] for hc in range(_H // 16)]

    def chunk_body(i, carry):
        chunk_id = i * _NW + wid

        @pl.when(chunk_id < _CHUNKS_TOTAL)
        def _():
            e0 = chunk_id * _CHUNK
            pltpu.sync_copy(src_hbm.at[pl.ds(e0, _CHUNK)], srcv)
            pltpu.sync_copy(dst_hbm.at[pl.ds(e0, _CHUNK)], dstv)
            g1 = pltpu.async_copy(xl_hbm.at[srcv], xlv, sem1)
            g2 = pltpu.async_copy(xr_hbm.at[dstv], xrv, sem2)
            g1.wait()
            g2.wait()

            def group_body(g, gcarry):
                dvec = dstv[pl.ds(g * 16, 16)]
                lanes = lax.iota(jnp.int32, 16)
                lane0 = lanes == 0
                avec = jnp.zeros((16,), jnp.float32)
                for l in range(16):
                    b = g * 16 + l
                    acc = jnp.zeros((16,), jnp.float32)
                    xl_regs = []
                    for hc in range(_H // 16):
                        xl = xlv[b, pl.ds(hc * 16, 16)]
                        xl_regs.append(xl)
                        t = xl + xrv[b, pl.ds(hc * 16, 16)]
                        t = jnp.maximum(t, 0.2 * t)
                        acc = acc + t * att_regs[hc]
                    for sh in (8, 4, 2, 1):
                        acc = acc + _lane_gather(acc, (lanes + sh) & 15)
                    a = jnp.exp(acc)
                    for hc in range(_H // 16):
                        xlv[b, pl.ds(hc * 16, 16)] = xl_regs[hc] * a
                    avec = jnp.where(lanes == l, a, avec)
                for l in range(16):
                    di = dvec[l]
                    al = _lane_gather(avec, jnp.full((16,), l, jnp.int32))
                    dval = denomv[pl.ds(di, 16)]
                    denomv[pl.ds(di, 16)] = (
                        dval + jnp.where(lane0, al, 0.0))
                return gcarry

            lax.fori_loop(0, _CHUNK // 16, group_body, 0)
            pltpu.sync_copy(xlv, accsh.at[dstv], add=True)

        return carry

    lax.fori_loop(0, _ITERS, chunk_body, 0)

    pltpu.sync_copy(denomv, den_hbm.at[cid, sid])
    plsc.subcore_barrier()

    @pl.when(sid == 0)
    def _():
        pltpu.sync_copy(accsh, feat_hbm.at[cid])


def _proj_body(x_ref, wl_ref, bl_ref, wr_ref, br_ref, xl_ref, xr_ref):
    x = x_ref[...]
    cdims = (((1,), (1,)), ((), ()))
    xl_ref[...] = (
        lax.dot_general(x, wl_ref[...], cdims,
                        preferred_element_type=jnp.float32) + bl_ref[...])
    xr_ref[...] = (
        lax.dot_general(x, wr_ref[...], cdims,
                        preferred_element_type=jnp.float32) + br_ref[...])


def _project(x, wl, bl, wr, br):
    blk = 1000
    grid = _N // blk
    return pl.pallas_call(
        _proj_body,
        grid=(grid,),
        in_specs=[
            pl.BlockSpec((blk, _H), lambda i: (i, 0)),
            pl.BlockSpec((_H, _H), lambda i: (0, 0)),
            pl.BlockSpec((1, _H), lambda i: (0, 0)),
            pl.BlockSpec((_H, _H), lambda i: (0, 0)),
            pl.BlockSpec((1, _H), lambda i: (0, 0)),
        ],
        out_specs=[
            pl.BlockSpec((blk, _H), lambda i: (i, 0)),
            pl.BlockSpec((blk, _H), lambda i: (i, 0)),
        ],
        out_shape=[
            jax.ShapeDtypeStruct((_N, _H), jnp.float32),
            jax.ShapeDtypeStruct((_N, _H), jnp.float32),
        ],
    )(x, wl, bl.reshape(1, _H), wr, br.reshape(1, _H))


def _den_col(d, ones):
    return lax.dot_general(d, ones, (((1,), (0,)), ((), ())),
                           preferred_element_type=jnp.float32)


def _mid_body(a0_ref, a1_ref, d_ref, ones_ref, b1_ref, wl_ref, bl_ref,
              wr_ref, br_ref, h_ref, xl_ref, xr_ref):
    num = a0_ref[...] + a1_ref[...]
    den = _den_col(d_ref[...], ones_ref[...])
    h = jnp.maximum(num / (den + 1e-16) + b1_ref[...], 0.0)
    h_ref[...] = h
    cdims = (((1,), (1,)), ((), ()))
    xl_ref[...] = (
        lax.dot_general(h, wl_ref[...], cdims,
                        preferred_element_type=jnp.float32) + bl_ref[...])
    xr_ref[...] = (
        lax.dot_general(h, wr_ref[...], cdims,
                        preferred_element_type=jnp.float32) + br_ref[...])


def _mid_stage(feat, den, ones, b1, wl2, bl2, wr2, br2):
    blk = 1000
    grid = _N // blk
    return pl.pallas_call(
        _mid_body,
        grid=(grid,),
        in_specs=[
            pl.BlockSpec((blk, _H), lambda i: (i, 0)),
            pl.BlockSpec((blk, _H), lambda i: (i, 0)),
            pl.BlockSpec((blk, _NW), lambda i: (i, 0)),
            pl.BlockSpec((_NW, 1), lambda i: (0, 0)),
            pl.BlockSpec((1, _H), lambda i: (0, 0)),
            pl.BlockSpec((_H, _H), lambda i: (0, 0)),
            pl.BlockSpec((1, _H), lambda i: (0, 0)),
            pl.BlockSpec((_H, _H), lambda i: (0, 0)),
            pl.BlockSpec((1, _H), lambda i: (0, 0)),
        ],
        out_specs=[
            pl.BlockSpec((blk, _H), lambda i: (i, 0)),
            pl.BlockSpec((blk, _H), lambda i: (i, 0)),
            pl.BlockSpec((blk, _H), lambda i: (i, 0)),
        ],
        out_shape=[
            jax.ShapeDtypeStruct((_N, _H), jnp.float32),
            jax.ShapeDtypeStruct((_N, _H), jnp.float32),
            jax.ShapeDtypeStruct((_N, _H), jnp.float32),
        ],
    )(feat[0], feat[1], den, ones, b1.reshape(1, _H), wl2,
      bl2.reshape(1, _H), wr2, br2.reshape(1, _H))


def _final_body(h_ref, a0_ref, a1_ref, d_ref, ones_ref, b2_ref, wlin_ref,
                blin_ref, out_ref):
    num = a0_ref[...] + a1_ref[...]
    den = _den_col(d_ref[...], ones_ref[...])
    h2 = num / (den + 1e-16) + b2_ref[...]
    hf = h_ref[...] + h2
    cdims = (((1,), (1,)), ((), ()))
    logits = (
        lax.dot_general(hf, wlin_ref[...], cdims,
                        preferred_element_type=jnp.float32) + blin_ref[...])
    m = jnp.max(logits, axis=1, keepdims=True)
    z = logits - m
    lse = jnp.log(jnp.sum(jnp.exp(z), axis=1, keepdims=True))
    out_ref[...] = z - lse


def _final_stage(h, feat, den, ones, b2, wlin, blin):
    blk = 1000
    grid = _N // blk
    c = wlin.shape[0]
    return pl.pallas_call(
        _final_body,
        grid=(grid,),
        in_specs=[
            pl.BlockSpec((blk, _H), lambda i: (i, 0)),
            pl.BlockSpec((blk, _H), lambda i: (i, 0)),
            pl.BlockSpec((blk, _H), lambda i: (i, 0)),
            pl.BlockSpec((blk, _NW), lambda i: (i, 0)),
            pl.BlockSpec((_NW, 1), lambda i: (0, 0)),
            pl.BlockSpec((1, _H), lambda i: (0, 0)),
            pl.BlockSpec((c, _H), lambda i: (0, 0)),
            pl.BlockSpec((1, c), lambda i: (0, 0)),
        ],
        out_specs=pl.BlockSpec((blk, c), lambda i: (i, 0)),
        out_shape=jax.ShapeDtypeStruct((_N, c), jnp.float32),
    )(h, feat[0], feat[1], den, ones, b2.reshape(1, _H), wlin,
      blin.reshape(1, -1))


def kernel(x, edge_index, Wl1, bl1, Wr1, br1, att1, b1,
           Wl2, bl2, Wr2, br2, att2, b2, Wlin, blin):
    src = edge_index[0]
    dst = edge_index[1]
    zeros = jnp.zeros((_N, _H), jnp.float32)
    ones = jnp.ones((_NW, 1), jnp.float32)

    xl1, xr1 = _project(x, Wl1, bl1, Wr1, br1)
    feat1, den1 = _edge_pass(xl1, xr1, src, dst, att1, zeros)
    h, xl2, xr2 = _mid_stage(feat1, den1.reshape(_NW, _NP).T, ones,
                             b1, Wl2, bl2, Wr2, br2)
    feat2, den2 = _edge_pass(xl2, xr2, src, dst, att2, zeros)
    out = _final_stage(h, feat2, den2.reshape(_NW, _NP).T, ones,
                       b2, Wlin, blin)
    return (out, edge_index)


# speedup vs baseline: 17.7273x; 1.4363x over previous
import functools

import jax
import jax.numpy as jnp
from jax import lax
from jax.experimental import pallas as pl
from jax.experimental.pallas import tpu as pltpu
from jax.experimental.pallas import tpu_sc as plsc

_N = 10000
_E = 320000
_H = 128
_NP = 10240
_CHUNK = 64
_NC = 2
_NS = 16
_NW = _NC * _NS
_CHUNKS_TOTAL = _E // _CHUNK
_WCHUNKS = (_CHUNKS_TOTAL + _NW - 1) // _NW
_SUP = 8
_NSUP = (_WCHUNKS + _SUP - 1) // _SUP
_EPAD = _NW * _NSUP * _SUP * _CHUNK

_sc_mesh = plsc.VectorSubcoreMesh(
    core_axis_name="c", subcore_axis_name="s", num_cores=_NC)

_GATHER_DNUMS = lax.GatherDimensionNumbers(
    offset_dims=(), collapsed_slice_dims=(0,), start_index_map=(0,))


def _lane_gather(v, idx):
    return lax.gather(
        v, idx[:, None], _GATHER_DNUMS, slice_sizes=(1,),
        mode=lax.GatherScatterMode.PROMISE_IN_BOUNDS)


@functools.partial(
    pl.kernel,
    mesh=_sc_mesh,
    out_type=[
        jax.ShapeDtypeStruct((_NC, _N, _H), jnp.float32),
        jax.ShapeDtypeStruct((_NC, _NS, _NP), jnp.float32),
    ],
    scratch_types=[
        pltpu.VMEM((_SUP * _CHUNK,), jnp.int32),
        pltpu.VMEM((_SUP * _CHUNK,), jnp.int32),
        pltpu.VMEM((_CHUNK,), jnp.int32),
        pltpu.VMEM((_CHUNK,), jnp.int32),
        pltpu.VMEM((_CHUNK, _H), jnp.float32),
        pltpu.VMEM((_CHUNK, _H), jnp.float32),
        pltpu.VMEM((_CHUNK, _H), jnp.float32),
        pltpu.VMEM((_CHUNK, _H), jnp.float32),
        pltpu.VMEM((_H,), jnp.float32),
        pltpu.VMEM((_NP,), jnp.float32),
        pltpu.VMEM_SHARED((_N, _H), jnp.float32),
        pltpu.SemaphoreType.DMA,
        pltpu.SemaphoreType.DMA,
        pltpu.SemaphoreType.DMA,
    ],
)
def _edge_pass(xl_hbm, xr_hbm, src_hbm, dst_hbm, att_hbm, zeros_hbm,
               feat_hbm, den_hbm, srci, dsti, dstc0, dstc1, xlv0, xlv1,
               xrv0, xrv1, attv, denomv, accsh, isem, gsem0, gsem1):
    cid = lax.axis_index("c")
    sid = lax.axis_index("s")

    @pl.when(sid == 0)
    def _():
        pltpu.sync_copy(zeros_hbm, accsh)

    pltpu.sync_copy(att_hbm, attv)

    def zero_body(i, carry):
        denomv[pl.ds(i * 16, 16)] = jnp.zeros((16,), jnp.float32)
        return carry

    lax.fori_loop(0, _NP // 16, zero_body, 0)
    plsc.subcore_barrier()

    wid = sid * _NC + cid
    wid_start = wid * _WCHUNKS
    wid_end = jnp.minimum(wid_start + _WCHUNKS, _CHUNKS_TOTAL)
    att_regs = [attv[pl.ds(hc * 16, 16)] for hc in range(_H // 16)]
    lanes = lax.iota(jnp.int32, 16)
    lane0 = lanes == 0

    bufs = ((dstc0, xlv0, xrv0, gsem0), (dstc1, xlv1, xrv1, gsem1))

    def issue_gather(parity, slot, pred):
        dstc, xlv, xrv, gsem = bufs[parity]

        @pl.when(pred)
        def _():
            off = slot * _CHUNK
            for j in range(_CHUNK // 16):
                dstc[pl.ds(j * 16, 16)] = dsti[pl.ds(off + j * 16, 16)]
            pltpu.async_copy(xl_hbm.at[srci.at[pl.ds(off, _CHUNK)]],
                             xlv, gsem)
            pltpu.async_copy(xr_hbm.at[dstc], xrv, gsem)

    def process_chunk(parity, pred):
        dstc, xlv, xrv, gsem = bufs[parity]

        @pl.when(pred)
        def _():
            pltpu.make_async_copy(xl_hbm, xlv, gsem).wait()
            pltpu.make_async_copy(xr_hbm, xrv, gsem).wait()

            def group_body(g, gcarry):
                dvec = dstc[pl.ds(g * 16, 16)]
                avec = jnp.zeros((16,), jnp.float32)
                for l in range(16):
                    b = g * 16 + l
                    acc = jnp.zeros((16,), jnp.float32)
                    xl_regs = []
                    for hc in range(_H // 16):
                        xl = xlv[b, pl.ds(hc * 16, 16)]
                        xl_regs.append(xl)
                        t = xl + xrv[b, pl.ds(hc * 16, 16)]
                        t = jnp.maximum(t, 0.2 * t)
                        acc = acc + t * att_regs[hc]
                    for sh in (8, 4, 2, 1):
                        acc = acc + _lane_gather(acc, (lanes + sh) & 15)
                    a = jnp.exp(acc)
                    for hc in range(_H // 16):
                        xlv[b, pl.ds(hc * 16, 16)] = xl_regs[hc] * a
                    avec = jnp.where(lanes == l, a, avec)
                for l in range(16):
                    di = dvec[l]
                    al = _lane_gather(avec, jnp.full((16,), l, jnp.int32))
                    dval = denomv[pl.ds(di, 16)]
                    denomv[pl.ds(di, 16)] = (
                        dval + jnp.where(lane0, al, 0.0))
                return gcarry

            lax.fori_loop(0, _CHUNK // 16, group_body, 0)
            pltpu.sync_copy(xlv, accsh.at[dstc], add=True)

    def sup_body(s, carry):
        base_chunk = wid_start + s * _SUP
        base_e = base_chunk * _CHUNK
        i1 = pltpu.async_copy(src_hbm.at[pl.ds(base_e, _SUP * _CHUNK)],
                              srci, isem)
        i2 = pltpu.async_copy(dst_hbm.at[pl.ds(base_e, _SUP * _CHUNK)],
                              dsti, isem)
        i1.wait()
        i2.wait()
        issue_gather(0, 0, base_chunk < wid_end)

        def pair_body(p, pcarry):
            slot0 = p * 2
            in_sup = slot0 + 2 < _SUP
            issue_gather(1, slot0 + 1, base_chunk + slot0 + 1 < wid_end)
            process_chunk(0, base_chunk + slot0 < wid_end)
            issue_gather(0, slot0 + 2,
                         jnp.logical_and(
                             in_sup,
                             base_chunk + slot0 + 2 < wid_end))
            process_chunk(1, base_chunk + slot0 + 1 < wid_end)
            return pcarry

        lax.fori_loop(0, _SUP // 2, pair_body, 0)
        return carry

    lax.fori_loop(0, _NSUP, sup_body, 0)

    pltpu.sync_copy(denomv, den_hbm.at[cid, sid])
    plsc.subcore_barrier()

    @pl.when(sid == 0)
    def _():
        pltpu.sync_copy(accsh, feat_hbm.at[cid])


def _proj_body(x_ref, wl_ref, bl_ref, wr_ref, br_ref, xl_ref, xr_ref):
    x = x_ref[...]
    cdims = (((1,), (1,)), ((), ()))
    xl_ref[...] = (
        lax.dot_general(x, wl_ref[...], cdims,
                        preferred_element_type=jnp.float32) + bl_ref[...])
    xr_ref[...] = (
        lax.dot_general(x, wr_ref[...], cdims,
                        preferred_element_type=jnp.float32) + br_ref[...])


def _project(x, wl, bl, wr, br):
    blk = 1000
    grid = _N // blk
    return pl.pallas_call(
        _proj_body,
        grid=(grid,),
        in_specs=[
            pl.BlockSpec((blk, _H), lambda i: (i, 0)),
            pl.BlockSpec((_H, _H), lambda i: (0, 0)),
            pl.BlockSpec((1, _H), lambda i: (0, 0)),
            pl.BlockSpec((_H, _H), lambda i: (0, 0)),
            pl.BlockSpec((1, _H), lambda i: (0, 0)),
        ],
        out_specs=[
            pl.BlockSpec((blk, _H), lambda i: (i, 0)),
            pl.BlockSpec((blk, _H), lambda i: (i, 0)),
        ],
        out_shape=[
            jax.ShapeDtypeStruct((_N, _H), jnp.float32),
            jax.ShapeDtypeStruct((_N, _H), jnp.float32),
        ],
    )(x, wl, bl.reshape(1, _H), wr, br.reshape(1, _H))


def _den_col(d, ones):
    return lax.dot_general(d, ones, (((1,), (0,)), ((), ())),
                           preferred_element_type=jnp.float32)


def _mid_body(a0_ref, a1_ref, d_ref, ones_ref, b1_ref, wl_ref, bl_ref,
              wr_ref, br_ref, h_ref, xl_ref, xr_ref):
    num = a0_ref[...] + a1_ref[...]
    den = _den_col(d_ref[...], ones_ref[...])
    h = jnp.maximum(num / (den + 1e-16) + b1_ref[...], 0.0)
    h_ref[...] = h
    cdims = (((1,), (1,)), ((), ()))
    xl_ref[...] = (
        lax.dot_general(h, wl_ref[...], cdims,
                        preferred_element_type=jnp.float32) + bl_ref[...])
    xr_ref[...] = (
        lax.dot_general(h, wr_ref[...], cdims,
                        preferred_element_type=jnp.float32) + br_ref[...])


def _mid_stage(feat, den, ones, b1, wl2, bl2, wr2, br2):
    blk = 1000
    grid = _N // blk
    return pl.pallas_call(
        _mid_body,
        grid=(grid,),
        in_specs=[
            pl.BlockSpec((blk, _H), lambda i: (i, 0)),
            pl.BlockSpec((blk, _H), lambda i: (i, 0)),
            pl.BlockSpec((blk, _NW), lambda i: (i, 0)),
            pl.BlockSpec((_NW, 1), lambda i: (0, 0)),
            pl.BlockSpec((1, _H), lambda i: (0, 0)),
            pl.BlockSpec((_H, _H), lambda i: (0, 0)),
            pl.BlockSpec((1, _H), lambda i: (0, 0)),
            pl.BlockSpec((_H, _H), lambda i: (0, 0)),
            pl.BlockSpec((1, _H), lambda i: (0, 0)),
        ],
        out_specs=[
            pl.BlockSpec((blk, _H), lambda i: (i, 0)),
            pl.BlockSpec((blk, _H), lambda i: (i, 0)),
            pl.BlockSpec((blk, _H), lambda i: (i, 0)),
        ],
        out_shape=[
            jax.ShapeDtypeStruct((_N, _H), jnp.float32),
            jax.ShapeDtypeStruct((_N, _H), jnp.float32),
            jax.ShapeDtypeStruct((_N, _H), jnp.float32),
        ],
    )(feat[0], feat[1], den, ones, b1.reshape(1, _H), wl2,
      bl2.reshape(1, _H), wr2, br2.reshape(1, _H))


def _final_body(h_ref, a0_ref, a1_ref, d_ref, ones_ref, b2_ref, wlin_ref,
                blin_ref, out_ref):
    num = a0_ref[...] + a1_ref[...]
    den = _den_col(d_ref[...], ones_ref[...])
    h2 = num / (den + 1e-16) + b2_ref[...]
    hf = h_ref[...] + h2
    cdims = (((1,), (1,)), ((), ()))
    logits = (
        lax.dot_general(hf, wlin_ref[...], cdims,
                        preferred_element_type=jnp.float32) + blin_ref[...])
    m = jnp.max(logits, axis=1, keepdims=True)
    z = logits - m
    lse = jnp.log(jnp.sum(jnp.exp(z), axis=1, keepdims=True))
    out_ref[...] = z - lse


def _final_stage(h, feat, den, ones, b2, wlin, blin):
    blk = 1000
    grid = _N // blk
    c = wlin.shape[0]
    return pl.pallas_call(
        _final_body,
        grid=(grid,),
        in_specs=[
            pl.BlockSpec((blk, _H), lambda i: (i, 0)),
            pl.BlockSpec((blk, _H), lambda i: (i, 0)),
            pl.BlockSpec((blk, _H), lambda i: (i, 0)),
            pl.BlockSpec((blk, _NW), lambda i: (i, 0)),
            pl.BlockSpec((_NW, 1), lambda i: (0, 0)),
            pl.BlockSpec((1, _H), lambda i: (0, 0)),
            pl.BlockSpec((c, _H), lambda i: (0, 0)),
            pl.BlockSpec((1, c), lambda i: (0, 0)),
        ],
        out_specs=pl.BlockSpec((blk, c), lambda i: (i, 0)),
        out_shape=jax.ShapeDtypeStruct((_N, c), jnp.float32),
    )(h, feat[0], feat[1], den, ones, b2.reshape(1, _H), wlin,
      blin.reshape(1, -1))


def kernel(x, edge_index, Wl1, bl1, Wr1, br1, att1, b1,
           Wl2, bl2, Wr2, br2, att2, b2, Wlin, blin):
    pad = _EPAD - _E
    src = jnp.pad(edge_index[0], (0, pad))
    dst = jnp.pad(edge_index[1], (0, pad))
    zeros = jnp.zeros((_N, _H), jnp.float32)
    ones = jnp.ones((_NW, 1), jnp.float32)

    xl1, xr1 = _project(x, Wl1, bl1, Wr1, br1)
    feat1, den1 = _edge_pass(xl1, xr1, src, dst, att1, zeros)
    h, xl2, xr2 = _mid_stage(feat1, den1.reshape(_NW, _NP).T, ones,
                             b1, Wl2, bl2, Wr2, br2)
    feat2, den2 = _edge_pass(xl2, xr2, src, dst, att2, zeros)
    out = _final_stage(h, feat2, den2.reshape(_NW, _NP).T, ones,
                       b2, Wlin, blin)
    return (out, edge_index)


# speedup vs baseline: 18.3629x; 1.0359x over previous
import functools

import jax
import jax.numpy as jnp
from jax import lax
from jax.experimental import pallas as pl
from jax.experimental.pallas import tpu as pltpu
from jax.experimental.pallas import tpu_sc as plsc

_N = 10000
_E = 320000
_H = 128
_NP = 10240
_CHUNK = 64
_NC = 2
_NS = 16
_NW = _NC * _NS
_CHUNKS_TOTAL = _E // _CHUNK
_WCHUNKS = (_CHUNKS_TOTAL + _NW - 1) // _NW
_SUP = 8
_NSUP = (_WCHUNKS + _SUP - 1) // _SUP
_EPAD = _NW * _NSUP * _SUP * _CHUNK

_sc_mesh = plsc.VectorSubcoreMesh(
    core_axis_name="c", subcore_axis_name="s", num_cores=_NC)

_GATHER_DNUMS = lax.GatherDimensionNumbers(
    offset_dims=(), collapsed_slice_dims=(0,), start_index_map=(0,))


def _lane_gather(v, idx):
    return lax.gather(
        v, idx[:, None], _GATHER_DNUMS, slice_sizes=(1,),
        mode=lax.GatherScatterMode.PROMISE_IN_BOUNDS)


@functools.partial(
    pl.kernel,
    mesh=_sc_mesh,
    out_type=[
        jax.ShapeDtypeStruct((_NC, _N, _H), jnp.float32),
        jax.ShapeDtypeStruct((_NC, _NS, _NP), jnp.float32),
    ],
    scratch_types=[
        pltpu.VMEM((_SUP * _CHUNK,), jnp.int32),
        pltpu.VMEM((_SUP * _CHUNK,), jnp.int32),
        pltpu.VMEM((_CHUNK,), jnp.int32),
        pltpu.VMEM((_CHUNK,), jnp.int32),
        pltpu.VMEM((_CHUNK, _H), jnp.float32),
        pltpu.VMEM((_CHUNK, _H), jnp.float32),
        pltpu.VMEM((_CHUNK, _H), jnp.float32),
        pltpu.VMEM((_CHUNK, _H), jnp.float32),
        pltpu.VMEM((_H,), jnp.float32),
        pltpu.VMEM((_NP,), jnp.float32),
        pltpu.VMEM_SHARED((_N, _H), jnp.float32),
        pltpu.SemaphoreType.DMA,
        pltpu.SemaphoreType.DMA,
        pltpu.SemaphoreType.DMA,
        pltpu.SemaphoreType.DMA,
        pltpu.SemaphoreType.DMA,
    ],
)
def _edge_pass(xl_hbm, xr_hbm, src_hbm, dst_hbm, att_hbm, zeros_hbm,
               feat_hbm, den_hbm, srci, dsti, dstc0, dstc1, xlv0, xlv1,
               xrv0, xrv1, attv, denomv, accsh, isem, gsem0, gsem1,
               ssem0, ssem1):
    cid = lax.axis_index("c")
    sid = lax.axis_index("s")

    @pl.when(sid == 0)
    def _():
        pltpu.sync_copy(zeros_hbm, accsh)

    pltpu.sync_copy(att_hbm, attv)

    def zero_body(i, carry):
        denomv[pl.ds(i * 16, 16)] = jnp.zeros((16,), jnp.float32)
        return carry

    lax.fori_loop(0, _NP // 16, zero_body, 0)
    plsc.subcore_barrier()

    wid = sid * _NC + cid
    wid_start = wid * _WCHUNKS
    wid_end = jnp.minimum(wid_start + _WCHUNKS, _CHUNKS_TOTAL)
    att_regs = [attv[pl.ds(hc * 16, 16)] for hc in range(_H // 16)]
    lanes = lax.iota(jnp.int32, 16)
    lane0 = lanes == 0

    bufs = ((dstc0, xlv0, xrv0, gsem0, ssem0),
            (dstc1, xlv1, xrv1, gsem1, ssem1))

    def issue_gather(parity, slot, pred, drain_pred):
        dstc, xlv, xrv, gsem, ssem = bufs[parity]

        @pl.when(drain_pred)
        def _():
            pltpu.make_async_copy(xl_hbm, xlv, ssem).wait()

        @pl.when(pred)
        def _():
            off = slot * _CHUNK
            for j in range(_CHUNK // 16):
                dstc[pl.ds(j * 16, 16)] = dsti[pl.ds(off + j * 16, 16)]
            pltpu.async_copy(xl_hbm.at[srci.at[pl.ds(off, _CHUNK)]],
                             xlv, gsem)
            pltpu.async_copy(xr_hbm.at[dstc], xrv, gsem)

    def process_chunk(parity, pred):
        dstc, xlv, xrv, gsem, ssem = bufs[parity]

        @pl.when(pred)
        def _():
            pltpu.make_async_copy(xl_hbm, xlv, gsem).wait()
            pltpu.make_async_copy(xr_hbm, xrv, gsem).wait()

            def group_body(g, gcarry):
                dvec = dstc[pl.ds(g * 16, 16)]
                avec = jnp.zeros((16,), jnp.float32)
                for l in range(16):
                    b = g * 16 + l
                    acc = jnp.zeros((16,), jnp.float32)
                    xl_regs = []
                    for hc in range(_H // 16):
                        xl = xlv[b, pl.ds(hc * 16, 16)]
                        xl_regs.append(xl)
                        t = xl + xrv[b, pl.ds(hc * 16, 16)]
                        t = jnp.maximum(t, 0.2 * t)
                        acc = acc + t * att_regs[hc]
                    for sh in (8, 4, 2, 1):
                        acc = acc + _lane_gather(acc, (lanes + sh) & 15)
                    a = jnp.exp(acc)
                    for hc in range(_H // 16):
                        xlv[b, pl.ds(hc * 16, 16)] = xl_regs[hc] * a
                    avec = jnp.where(lanes == l, a, avec)
                for l in range(16):
                    di = dvec[l]
                    al = _lane_gather(avec, jnp.full((16,), l, jnp.int32))
                    dval = denomv[pl.ds(di, 16)]
                    denomv[pl.ds(di, 16)] = (
                        dval + jnp.where(lane0, al, 0.0))
                return gcarry

            lax.fori_loop(0, _CHUNK // 16, group_body, 0)
            pltpu.async_copy(xlv, accsh.at[dstc], ssem, add=True)

    def sup_body(s, carry):
        base_chunk = wid_start + s * _SUP
        base_e = base_chunk * _CHUNK
        i1 = pltpu.async_copy(src_hbm.at[pl.ds(base_e, _SUP * _CHUNK)],
                              srci, isem)
        i2 = pltpu.async_copy(dst_hbm.at[pl.ds(base_e, _SUP * _CHUNK)],
                              dsti, isem)
        i1.wait()
        i2.wait()
        g0 = base_chunk < wid_end
        issue_gather(0, 0, g0, jnp.logical_and(g0, s > 0))

        def pair_body(p, pcarry):
            slot0 = p * 2
            in_sup = slot0 + 2 < _SUP
            g1 = base_chunk + slot0 + 1 < wid_end
            g2 = jnp.logical_and(in_sup,
                                 base_chunk + slot0 + 2 < wid_end)
            issue_gather(1, slot0 + 1, g1,
                         jnp.logical_and(
                             g1, jnp.logical_or(s > 0, p > 0)))
            process_chunk(0, base_chunk + slot0 < wid_end)
            issue_gather(0, slot0 + 2, g2, g2)
            process_chunk(1, g1)
            return pcarry

        lax.fori_loop(0, _SUP // 2, pair_body, 0)
        return carry

    lax.fori_loop(0, _NSUP, sup_body, 0)

    pltpu.make_async_copy(xl_hbm, xlv0, ssem0).wait()
    pltpu.make_async_copy(xl_hbm, xlv1, ssem1).wait()

    pltpu.sync_copy(denomv, den_hbm.at[cid, sid])
    plsc.subcore_barrier()

    @pl.when(sid == 0)
    def _():
        pltpu.sync_copy(accsh, feat_hbm.at[cid])


def _proj_body(x_ref, wl_ref, bl_ref, wr_ref, br_ref, xl_ref, xr_ref):
    x = x_ref[...]
    cdims = (((1,), (1,)), ((), ()))
    xl_ref[...] = (
        lax.dot_general(x, wl_ref[...], cdims,
                        preferred_element_type=jnp.float32) + bl_ref[...])
    xr_ref[...] = (
        lax.dot_general(x, wr_ref[...], cdims,
                        preferred_element_type=jnp.float32) + br_ref[...])


def _project(x, wl, bl, wr, br):
    blk = 1000
    grid = _N // blk
    return pl.pallas_call(
        _proj_body,
        grid=(grid,),
        in_specs=[
            pl.BlockSpec((blk, _H), lambda i: (i, 0)),
            pl.BlockSpec((_H, _H), lambda i: (0, 0)),
            pl.BlockSpec((1, _H), lambda i: (0, 0)),
            pl.BlockSpec((_H, _H), lambda i: (0, 0)),
            pl.BlockSpec((1, _H), lambda i: (0, 0)),
        ],
        out_specs=[
            pl.BlockSpec((blk, _H), lambda i: (i, 0)),
            pl.BlockSpec((blk, _H), lambda i: (i, 0)),
        ],
        out_shape=[
            jax.ShapeDtypeStruct((_N, _H), jnp.float32),
            jax.ShapeDtypeStruct((_N, _H), jnp.float32),
        ],
    )(x, wl, bl.reshape(1, _H), wr, br.reshape(1, _H))


def _den_col(d, ones):
    return lax.dot_general(d, ones, (((1,), (0,)), ((), ())),
                           preferred_element_type=jnp.float32)


def _mid_body(a0_ref, a1_ref, d_ref, ones_ref, b1_ref, wl_ref, bl_ref,
              wr_ref, br_ref, h_ref, xl_ref, xr_ref):
    num = a0_ref[...] + a1_ref[...]
    den = _den_col(d_ref[...], ones_ref[...])
    h = jnp.maximum(num / (den + 1e-16) + b1_ref[...], 0.0)
    h_ref[...] = h
    cdims = (((1,), (1,)), ((), ()))
    xl_ref[...] = (
        lax.dot_general(h, wl_ref[...], cdims,
                        preferred_element_type=jnp.float32) + bl_ref[...])
    xr_ref[...] = (
        lax.dot_general(h, wr_ref[...], cdims,
                        preferred_element_type=jnp.float32) + br_ref[...])


def _mid_stage(feat, den, ones, b1, wl2, bl2, wr2, br2):
    blk = 1000
    grid = _N // blk
    return pl.pallas_call(
        _mid_body,
        grid=(grid,),
        in_specs=[
            pl.BlockSpec((blk, _H), lambda i: (i, 0)),
            pl.BlockSpec((blk, _H), lambda i: (i, 0)),
            pl.BlockSpec((blk, _NW), lambda i: (i, 0)),
            pl.BlockSpec((_NW, 1), lambda i: (0, 0)),
            pl.BlockSpec((1, _H), lambda i: (0, 0)),
            pl.BlockSpec((_H, _H), lambda i: (0, 0)),
            pl.BlockSpec((1, _H), lambda i: (0, 0)),
            pl.BlockSpec((_H, _H), lambda i: (0, 0)),
            pl.BlockSpec((1, _H), lambda i: (0, 0)),
        ],
        out_specs=[
            pl.BlockSpec((blk, _H), lambda i: (i, 0)),
            pl.BlockSpec((blk, _H), lambda i: (i, 0)),
            pl.BlockSpec((blk, _H), lambda i: (i, 0)),
        ],
        out_shape=[
            jax.ShapeDtypeStruct((_N, _H), jnp.float32),
            jax.ShapeDtypeStruct((_N, _H), jnp.float32),
            jax.ShapeDtypeStruct((_N, _H), jnp.float32),
        ],
    )(feat[0], feat[1], den, ones, b1.reshape(1, _H), wl2,
      bl2.reshape(1, _H), wr2, br2.reshape(1, _H))


def _final_body(h_ref, a0_ref, a1_ref, d_ref, ones_ref, b2_ref, wlin_ref,
                blin_ref, out_ref):
    num = a0_ref[...] + a1_ref[...]
    den = _den_col(d_ref[...], ones_ref[...])
    h2 = num / (den + 1e-16) + b2_ref[...]
    hf = h_ref[...] + h2
    cdims = (((1,), (1,)), ((), ()))
    logits = (
        lax.dot_general(hf, wlin_ref[...], cdims,
                        preferred_element_type=jnp.float32) + blin_ref[...])
    m = jnp.max(logits, axis=1, keepdims=True)
    z = logits - m
    lse = jnp.log(jnp.sum(jnp.exp(z), axis=1, keepdims=True))
    out_ref[...] = z - lse


def _final_stage(h, feat, den, ones, b2, wlin, blin):
    blk = 1000
    grid = _N // blk
    c = wlin.shape[0]
    return pl.pallas_call(
        _final_body,
        grid=(grid,),
        in_specs=[
            pl.BlockSpec((blk, _H), lambda i: (i, 0)),
            pl.BlockSpec((blk, _H), lambda i: (i, 0)),
            pl.BlockSpec((blk, _H), lambda i: (i, 0)),
            pl.BlockSpec((blk, _NW), lambda i: (i, 0)),
            pl.BlockSpec((_NW, 1), lambda i: (0, 0)),
            pl.BlockSpec((1, _H), lambda i: (0, 0)),
            pl.BlockSpec((c, _H), lambda i: (0, 0)),
            pl.BlockSpec((1, c), lambda i: (0, 0)),
        ],
        out_specs=pl.BlockSpec((blk, c), lambda i: (i, 0)),
        out_shape=jax.ShapeDtypeStruct((_N, c), jnp.float32),
    )(h, feat[0], feat[1], den, ones, b2.reshape(1, _H), wlin,
      blin.reshape(1, -1))


def kernel(x, edge_index, Wl1, bl1, Wr1, br1, att1, b1,
           Wl2, bl2, Wr2, br2, att2, b2, Wlin, blin):
    pad = _EPAD - _E
    src = jnp.pad(edge_index[0], (0, pad))
    dst = jnp.pad(edge_index[1], (0, pad))
    zeros = jnp.zeros((_N, _H), jnp.float32)
    ones = jnp.ones((_NW, 1), jnp.float32)

    xl1, xr1 = _project(x, Wl1, bl1, Wr1, br1)
    feat1, den1 = _edge_pass(xl1, xr1, src, dst, att1, zeros)
    h, xl2, xr2 = _mid_stage(feat1, den1.reshape(_NW, _NP).T, ones,
                             b1, Wl2, bl2, Wr2, br2)
    feat2, den2 = _edge_pass(xl2, xr2, src, dst, att2, zeros)
    out = _final_stage(h, feat2, den2.reshape(_NW, _NP).T, ones,
                       b2, Wlin, blin)
    return (out, edge_index)


# speedup vs baseline: 19.2907x; 1.0505x over previous
import functools

import jax
import jax.numpy as jnp
from jax import lax
from jax.experimental import pallas as pl
from jax.experimental.pallas import tpu as pltpu
from jax.experimental.pallas import tpu_sc as plsc

_N = 10000
_E = 320000
_H = 128
_NP = 10240
_CHUNK = 64
_NC = 2
_NS = 16
_NW = _NC * _NS
_CHUNKS_TOTAL = _E // _CHUNK
_WCHUNKS = (_CHUNKS_TOTAL + _NW - 1) // _NW
_SUP = 16
_NSUP = (_WCHUNKS + _SUP - 1) // _SUP
_EPAD = _NW * _NSUP * _SUP * _CHUNK

_sc_mesh = plsc.VectorSubcoreMesh(
    core_axis_name="c", subcore_axis_name="s", num_cores=_NC)

_GATHER_DNUMS = lax.GatherDimensionNumbers(
    offset_dims=(), collapsed_slice_dims=(0,), start_index_map=(0,))


def _lane_gather(v, idx):
    return lax.gather(
        v, idx[:, None], _GATHER_DNUMS, slice_sizes=(1,),
        mode=lax.GatherScatterMode.PROMISE_IN_BOUNDS)


@functools.partial(
    pl.kernel,
    mesh=_sc_mesh,
    out_type=[
        jax.ShapeDtypeStruct((_NC, _N, _H), jnp.float32),
        jax.ShapeDtypeStruct((_NC, _NS, _NP), jnp.float32),
    ],
    scratch_types=[
        pltpu.VMEM((_SUP * _CHUNK,), jnp.int32),
        pltpu.VMEM((_SUP * _CHUNK,), jnp.int32),
        pltpu.VMEM((_CHUNK,), jnp.int32),
        pltpu.VMEM((_CHUNK,), jnp.int32),
        pltpu.VMEM((_CHUNK, _H), jnp.float32),
        pltpu.VMEM((_CHUNK, _H), jnp.float32),
        pltpu.VMEM((_CHUNK, _H), jnp.float32),
        pltpu.VMEM((_CHUNK, _H), jnp.float32),
        pltpu.VMEM((_H,), jnp.float32),
        pltpu.VMEM((_NP,), jnp.float32),
        pltpu.VMEM_SHARED((_N, _H), jnp.float32),
        pltpu.SemaphoreType.DMA,
        pltpu.SemaphoreType.DMA,
        pltpu.SemaphoreType.DMA,
        pltpu.SemaphoreType.DMA,
        pltpu.SemaphoreType.DMA,
    ],
)
def _edge_pass(xl_hbm, xr_hbm, src_hbm, dst_hbm, att_hbm, zeros_hbm,
               feat_hbm, den_hbm, srci, dsti, dstc0, dstc1, xlv0, xlv1,
               xrv0, xrv1, attv, denomv, accsh, isem, gsem0, gsem1,
               ssem0, ssem1):
    cid = lax.axis_index("c")
    sid = lax.axis_index("s")

    @pl.when(sid == 0)
    def _():
        pltpu.sync_copy(zeros_hbm, accsh)

    pltpu.sync_copy(att_hbm, attv)

    def zero_body(i, carry):
        denomv[pl.ds(i * 16, 16)] = jnp.zeros((16,), jnp.float32)
        return carry

    lax.fori_loop(0, _NP // 16, zero_body, 0)
    plsc.subcore_barrier()

    wid = sid * _NC + cid
    wid_start = wid * _WCHUNKS
    wid_end = jnp.minimum(wid_start + _WCHUNKS, _CHUNKS_TOTAL)
    att_regs = [attv[pl.ds(hc * 16, 16)] for hc in range(_H // 16)]
    lanes = lax.iota(jnp.int32, 16)
    lane0 = lanes == 0

    bufs = ((dstc0, xlv0, xrv0, gsem0, ssem0),
            (dstc1, xlv1, xrv1, gsem1, ssem1))

    def issue_gather(parity, slot, pred, drain_pred):
        dstc, xlv, xrv, gsem, ssem = bufs[parity]

        @pl.when(drain_pred)
        def _():
            pltpu.make_async_copy(xl_hbm, xlv, ssem).wait()

        @pl.when(pred)
        def _():
            off = slot * _CHUNK
            for j in range(_CHUNK // 16):
                dstc[pl.ds(j * 16, 16)] = dsti[pl.ds(off + j * 16, 16)]
            pltpu.async_copy(xl_hbm.at[srci.at[pl.ds(off, _CHUNK)]],
                             xlv, gsem)
            pltpu.async_copy(xr_hbm.at[dstc], xrv, gsem)

    def process_chunk(parity, pred):
        dstc, xlv, xrv, gsem, ssem = bufs[parity]

        @pl.when(pred)
        def _():
            pltpu.make_async_copy(xl_hbm, xlv, gsem).wait()
            pltpu.make_async_copy(xr_hbm, xrv, gsem).wait()

            def group_body(g, gcarry):
                dvec = dstc[pl.ds(g * 16, 16)]
                avec = jnp.zeros((16,), jnp.float32)
                for l in range(16):
                    b = g * 16 + l
                    acc = jnp.zeros((16,), jnp.float32)
                    xl_regs = []
                    for hc in range(_H // 16):
                        xl = xlv[b, pl.ds(hc * 16, 16)]
                        xl_regs.append(xl)
                        t = xl + xrv[b, pl.ds(hc * 16, 16)]
                        t = jnp.maximum(t, 0.2 * t)
                        acc = acc + t * att_regs[hc]
                    for sh in (8, 4, 2, 1):
                        acc = acc + _lane_gather(acc, (lanes + sh) & 15)
                    a = jnp.exp(acc)
                    for hc in range(_H // 16):
                        xlv[b, pl.ds(hc * 16, 16)] = xl_regs[hc] * a
                    avec = jnp.where(lanes == l, a, avec)
                for l in range(16):
                    di = dvec[l]
                    al = _lane_gather(avec, jnp.full((16,), l, jnp.int32))
                    dval = denomv[pl.ds(di, 16)]
                    denomv[pl.ds(di, 16)] = (
                        dval + jnp.where(lane0, al, 0.0))
                return gcarry

            lax.fori_loop(0, _CHUNK // 16, group_body, 0)
            pltpu.async_copy(xlv, accsh.at[dstc], ssem, add=True)

    def sup_body(s, carry):
        base_chunk = wid_start + s * _SUP
        base_e = base_chunk * _CHUNK
        i1 = pltpu.async_copy(src_hbm.at[pl.ds(base_e, _SUP * _CHUNK)],
                              srci, isem)
        i2 = pltpu.async_copy(dst_hbm.at[pl.ds(base_e, _SUP * _CHUNK)],
                              dsti, isem)
        i1.wait()
        i2.wait()
        g0 = base_chunk < wid_end
        issue_gather(0, 0, g0, jnp.logical_and(g0, s > 0))

        def pair_body(p, pcarry):
            slot0 = p * 2
            in_sup = slot0 + 2 < _SUP
            g1 = base_chunk + slot0 + 1 < wid_end
            g2 = jnp.logical_and(in_sup,
                                 base_chunk + slot0 + 2 < wid_end)
            issue_gather(1, slot0 + 1, g1,
                         jnp.logical_and(
                             g1, jnp.logical_or(s > 0, p > 0)))
            process_chunk(0, base_chunk + slot0 < wid_end)
            issue_gather(0, slot0 + 2, g2, g2)
            process_chunk(1, g1)
            return pcarry

        lax.fori_loop(0, _SUP // 2, pair_body, 0)
        return carry

    lax.fori_loop(0, _NSUP, sup_body, 0)

    pltpu.make_async_copy(xl_hbm, xlv0, ssem0).wait()
    pltpu.make_async_copy(xl_hbm, xlv1, ssem1).wait()

    pltpu.sync_copy(denomv, den_hbm.at[cid, sid])
    plsc.subcore_barrier()

    @pl.when(sid == 0)
    def _():
        pltpu.sync_copy(accsh, feat_hbm.at[cid])


def _proj_body(x_ref, wl_ref, bl_ref, wr_ref, br_ref, xl_ref, xr_ref):
    x = x_ref[...]
    cdims = (((1,), (1,)), ((), ()))
    xl_ref[...] = (
        lax.dot_general(x, wl_ref[...], cdims,
                        preferred_element_type=jnp.float32) + bl_ref[...])
    xr_ref[...] = (
        lax.dot_general(x, wr_ref[...], cdims,
                        preferred_element_type=jnp.float32) + br_ref[...])


def _project(x, wl, bl, wr, br):
    blk = 1000
    grid = _N // blk
    return pl.pallas_call(
        _proj_body,
        grid=(grid,),
        in_specs=[
            pl.BlockSpec((blk, _H), lambda i: (i, 0)),
            pl.BlockSpec((_H, _H), lambda i: (0, 0)),
            pl.BlockSpec((1, _H), lambda i: (0, 0)),
            pl.BlockSpec((_H, _H), lambda i: (0, 0)),
            pl.BlockSpec((1, _H), lambda i: (0, 0)),
        ],
        out_specs=[
            pl.BlockSpec((blk, _H), lambda i: (i, 0)),
            pl.BlockSpec((blk, _H), lambda i: (i, 0)),
        ],
        out_shape=[
            jax.ShapeDtypeStruct((_N, _H), jnp.float32),
            jax.ShapeDtypeStruct((_N, _H), jnp.float32),
        ],
    )(x, wl, bl.reshape(1, _H), wr, br.reshape(1, _H))


def _den_col(d, ones):
    return lax.dot_general(d, ones, (((1,), (0,)), ((), ())),
                           preferred_element_type=jnp.float32)


def _mid_body(a0_ref, a1_ref, d_ref, ones_ref, b1_ref, wl_ref, bl_ref,
              wr_ref, br_ref, h_ref, xl_ref, xr_ref):
    num = a0_ref[...] + a1_ref[...]
    den = _den_col(d_ref[...], ones_ref[...])
    h = jnp.maximum(num / (den + 1e-16) + b1_ref[...], 0.0)
    h_ref[...] = h
    cdims = (((1,), (1,)), ((), ()))
    xl_ref[...] = (
        lax.dot_general(h, wl_ref[...], cdims,
                        preferred_element_type=jnp.float32) + bl_ref[...])
    xr_ref[...] = (
        lax.dot_general(h, wr_ref[...], cdims,
                        preferred_element_type=jnp.float32) + br_ref[...])


def _mid_stage(feat, den, ones, b1, wl2, bl2, wr2, br2):
    blk = 1000
    grid = _N // blk
    return pl.pallas_call(
        _mid_body,
        grid=(grid,),
        in_specs=[
            pl.BlockSpec((blk, _H), lambda i: (i, 0)),
            pl.BlockSpec((blk, _H), lambda i: (i, 0)),
            pl.BlockSpec((blk, _NW), lambda i: (i, 0)),
            pl.BlockSpec((_NW, 1), lambda i: (0, 0)),
            pl.BlockSpec((1, _H), lambda i: (0, 0)),
            pl.BlockSpec((_H, _H), lambda i: (0, 0)),
            pl.BlockSpec((1, _H), lambda i: (0, 0)),
            pl.BlockSpec((_H, _H), lambda i: (0, 0)),
            pl.BlockSpec((1, _H), lambda i: (0, 0)),
        ],
        out_specs=[
            pl.BlockSpec((blk, _H), lambda i: (i, 0)),
            pl.BlockSpec((blk, _H), lambda i: (i, 0)),
            pl.BlockSpec((blk, _H), lambda i: (i, 0)),
        ],
        out_shape=[
            jax.ShapeDtypeStruct((_N, _H), jnp.float32),
            jax.ShapeDtypeStruct((_N, _H), jnp.float32),
            jax.ShapeDtypeStruct((_N, _H), jnp.float32),
        ],
    )(feat[0], feat[1], den, ones, b1.reshape(1, _H), wl2,
      bl2.reshape(1, _H), wr2, br2.reshape(1, _H))


def _final_body(h_ref, a0_ref, a1_ref, d_ref, ones_ref, b2_ref, wlin_ref,
                blin_ref, out_ref):
    num = a0_ref[...] + a1_ref[...]
    den = _den_col(d_ref[...], ones_ref[...])
    h2 = num / (den + 1e-16) + b2_ref[...]
    hf = h_ref[...] + h2
    cdims = (((1,), (1,)), ((), ()))
    logits = (
        lax.dot_general(hf, wlin_ref[...], cdims,
                        preferred_element_type=jnp.float32) + blin_ref[...])
    m = jnp.max(logits, axis=1, keepdims=True)
    z = logits - m
    lse = jnp.log(jnp.sum(jnp.exp(z), axis=1, keepdims=True))
    out_ref[...] = z - lse


def _final_stage(h, feat, den, ones, b2, wlin, blin):
    blk = 1000
    grid = _N // blk
    c = wlin.shape[0]
    return pl.pallas_call(
        _final_body,
        grid=(grid,),
        in_specs=[
            pl.BlockSpec((blk, _H), lambda i: (i, 0)),
            pl.BlockSpec((blk, _H), lambda i: (i, 0)),
            pl.BlockSpec((blk, _H), lambda i: (i, 0)),
            pl.BlockSpec((blk, _NW), lambda i: (i, 0)),
            pl.BlockSpec((_NW, 1), lambda i: (0, 0)),
            pl.BlockSpec((1, _H), lambda i: (0, 0)),
            pl.BlockSpec((c, _H), lambda i: (0, 0)),
            pl.BlockSpec((1, c), lambda i: (0, 0)),
        ],
        out_specs=pl.BlockSpec((blk, c), lambda i: (i, 0)),
        out_shape=jax.ShapeDtypeStruct((_N, c), jnp.float32),
    )(h, feat[0], feat[1], den, ones, b2.reshape(1, _H), wlin,
      blin.reshape(1, -1))


def kernel(x, edge_index, Wl1, bl1, Wr1, br1, att1, b1,
           Wl2, bl2, Wr2, br2, att2, b2, Wlin, blin):
    pad = _EPAD - _E
    src = jnp.pad(edge_index[0], (0, pad))
    dst = jnp.pad(edge_index[1], (0, pad))
    zeros = jnp.zeros((_N, _H), jnp.float32)
    ones = jnp.ones((_NW, 1), jnp.float32)

    xl1, xr1 = _project(x, Wl1, bl1, Wr1, br1)
    feat1, den1 = _edge_pass(xl1, xr1, src, dst, att1, zeros)
    h, xl2, xr2 = _mid_stage(feat1, den1.reshape(_NW, _NP).T, ones,
                             b1, Wl2, bl2, Wr2, br2)
    feat2, den2 = _edge_pass(xl2, xr2, src, dst, att2, zeros)
    out = _final_stage(h, feat2, den2.reshape(_NW, _NP).T, ones,
                       b2, Wlin, blin)
    return (out, edge_index)
